# Initial kernel scaffold; baseline (speedup 1.0000x reference)
import jax
import jax.numpy as jnp
from jax.experimental import pallas as pl


def kernel(x, edge_index, edge_type, edge_weight, weights, bias):
    raise NotImplementedError("write your pallas kernel here")


# speedup vs baseline: 33.3351x; 33.3351x over previous
import functools

import jax
import jax.numpy as jnp
from jax import lax
from jax.experimental import pallas as pl
from jax.experimental.pallas import tpu as pltpu
from jax.experimental.pallas import tpu_sc as plsc

N_NODES = 10000
N_EDGES = 320000
D = 128
N_REL = 4

NC = 2
NS = 16
NW = NC * NS

K = 80
ROWS2D = N_EDGES // K
CH_T = N_EDGES // NW // K
BS = 25
HROWS = ROWS2D // NS
DROWS = 640

_f32 = jnp.float32
_i32 = jnp.int32


def _zero16():
    return jnp.zeros((16,), _f32)


def _rsqrt16(d):
    i = plsc.bitcast(d, _i32)
    i = 0x5F3759DF - lax.shift_right_logical(i, 1)
    y = plsc.bitcast(i, _f32)
    half_d = 0.5 * d
    for _ in range(3):
        y = y * (1.5 - half_d * y * y)
    return jnp.where(d > 0.5, y, jnp.zeros_like(y))


def _sc_body(row_hbm, col_hbm, typ_hbm, ew_hbm, table_hbm, out_hbm,
             row_s, col_s, typ_s, ew_s, dnv, idx2, buf0, buf1,
             sem0, sem1, deg_sh, acc_sh):
    cid = lax.axis_index("c")
    sid = lax.axis_index("s")
    wid = cid * NS + sid
    ones16 = jnp.ones((16,), _f32)

    def _z_dnv(i, _):
        dnv[i, :] = _zero16()
        return _
    lax.fori_loop(0, DROWS, _z_dnv, None)

    def _z_buf(i, _):
        for q in range(D // 16):
            buf0[i, pl.ds(q * 16, 16)] = _zero16()
        return _
    lax.fori_loop(0, K, _z_buf, None)

    def _z_idx(k, _):
        for q in range(8):
            idx2[k, pl.ds(q * 16, 16)] = (
                lax.iota(_i32, 16) + (k * 128 + q * 16))
        return _
    lax.fori_loop(0, 5, _z_idx, None)

    def _z_acc(k, _):
        pltpu.sync_copy(buf0, acc_sh.at[pl.ds(sid * 625 + k * K, K)])
        return _
    lax.fori_loop(0, 7, _z_acc, None)
    pltpu.sync_copy(buf0.at[pl.ds(0, 65)],
                    acc_sh.at[pl.ds(sid * 625 + 7 * K, 65)])

    @pl.when(sid == 0)
    def _():
        pltpu.sync_copy(dnv, deg_sh)

    plsc.subcore_barrier()

    def _hist_stage(st, _):
        pltpu.sync_copy(col_hbm.at[pl.ds(sid * HROWS + st * BS, BS)], col_s)

        def _hrow(j, _h):
            for q in range(K // 16):
                v = col_s[j, pl.ds(q * 16, 16)]
                plsc.addupdate_scatter(
                    dnv, [lax.shift_right_logical(v, 4),
                          jnp.bitwise_and(v, 15)], ones16)
            return _h
        lax.fori_loop(0, BS, _hrow, None)
        return _
    lax.fori_loop(0, HROWS // BS, _hist_stage, None)

    for k in range(5):
        pltpu.sync_copy(dnv.at[pl.ds(k * 128, 128)],
                        deg_sh.at[idx2.at[k]], add=True)
    plsc.subcore_barrier()

    pltpu.sync_copy(deg_sh, dnv)

    def _dinv(i, _):
        dnv[i, :] = _rsqrt16(dnv[i, :])
        return _
    lax.fori_loop(0, DROWS, _dinv, None)

    def _scale(buf, ref, j):
        def _sgrp(q, _):
            sv16 = ref[j, pl.ds(q * 16, 16)]
            for l in range(16):
                sv = jnp.full((16,), sv16[l], _f32)
                i = q * 16 + l
                for p in range(D // 16):
                    sl = pl.ds(p * 16, 16)
                    buf[i, sl] = buf[i, sl] * sv
            return _
        lax.fori_loop(0, K // 16, _sgrp, None)

    def _block(b, _):
        base = wid * CH_T + b * BS
        pltpu.sync_copy(row_hbm.at[pl.ds(base, BS)], row_s)
        pltpu.sync_copy(col_hbm.at[pl.ds(base, BS)], col_s)
        pltpu.sync_copy(typ_hbm.at[pl.ds(base, BS)], typ_s)
        pltpu.sync_copy(ew_hbm.at[pl.ds(base, BS)], ew_s)

        def _prep(j, _p):
            for q in range(K // 16):
                sl = pl.ds(q * 16, 16)
                r = row_s[j, sl]
                c = col_s[j, sl]
                t = typ_s[j, sl]
                w = ew_s[j, sl]
                dr = plsc.load_gather(
                    dnv, [lax.shift_right_logical(r, 4),
                          jnp.bitwise_and(r, 15)])
                dc = plsc.load_gather(
                    dnv, [lax.shift_right_logical(c, 4),
                          jnp.bitwise_and(c, 15)])
                ew_s[j, sl] = w * dr * dc
                col_s[j, sl] = t * N_NODES + c
            return _p
        lax.fori_loop(0, BS, _prep, None)

        def _pair(p, _p):
            j0 = 2 * p
            j1 = 2 * p + 1
            d0 = pltpu.async_copy(table_hbm.at[col_s.at[j0]], buf0, sem0)
            d1 = pltpu.async_copy(table_hbm.at[col_s.at[j1]], buf1, sem1)
            d0.wait()
            _scale(buf0, ew_s, j0)
            pltpu.sync_copy(buf0, acc_sh.at[row_s.at[j0]], add=True)
            d1.wait()
            _scale(buf1, ew_s, j1)
            pltpu.sync_copy(buf1, acc_sh.at[row_s.at[j1]], add=True)
            return _p
        lax.fori_loop(0, BS // 2, _pair, None)

        jl = BS - 1
        pltpu.async_copy(table_hbm.at[col_s.at[jl]], buf0, sem0).wait()
        _scale(buf0, ew_s, jl)
        pltpu.sync_copy(buf0, acc_sh.at[row_s.at[jl]], add=True)
        return _
    lax.fori_loop(0, CH_T // BS, _block, None)

    plsc.subcore_barrier()

    def _out(k, _):
        rbase = sid * 625 + k * K
        pltpu.sync_copy(acc_sh.at[pl.ds(rbase, K)], buf0)
        pltpu.sync_copy(buf0, out_hbm.at[pl.ds(cid * N_NODES + rbase, K)])
        return _
    lax.fori_loop(0, 7, _out, None)
    pltpu.sync_copy(acc_sh.at[pl.ds(sid * 625 + 7 * K, 65)],
                    buf0.at[pl.ds(0, 65)])
    pltpu.sync_copy(buf0.at[pl.ds(0, 65)],
                    out_hbm.at[pl.ds(cid * N_NODES + sid * 625 + 7 * K, 65)])


def _make_sc_kernel():
    mesh = plsc.VectorSubcoreMesh(core_axis_name="c", subcore_axis_name="s")

    return pl.kernel(
        _sc_body,
        out_type=jax.ShapeDtypeStruct((NC * N_NODES, D), _f32),
        mesh=mesh,
        compiler_params=pltpu.CompilerParams(
            needs_layout_passes=False, use_tc_tiling_on_sc=False),
        scratch_types=[
            pltpu.VMEM((BS, K), _i32),
            pltpu.VMEM((BS, K), _i32),
            pltpu.VMEM((BS, K), _i32),
            pltpu.VMEM((BS, K), _f32),
            pltpu.VMEM((DROWS, 16), _f32),
            pltpu.VMEM((5, 128), _i32),
            pltpu.VMEM((K, D), _f32),
            pltpu.VMEM((K, D), _f32),
            pltpu.SemaphoreType.DMA,
            pltpu.SemaphoreType.DMA,
            pltpu.VMEM_SHARED((DROWS, 16), _f32),
            pltpu.VMEM_SHARED((N_NODES, D), _f32),
        ],
    )


_sc_kernel_cache = []


def _get_sc_kernel():
    if not _sc_kernel_cache:
        _sc_kernel_cache.append(_make_sc_kernel())
    return _sc_kernel_cache[0]


def _mm_body(x_ref, w_ref, o_ref):
    o_ref[...] = jnp.dot(x_ref[...], w_ref[0],
                         preferred_element_type=_f32)


_MB = 1000


def _project(x, weights):
    return pl.pallas_call(
        _mm_body,
        grid=(N_REL, N_NODES // _MB),
        in_specs=[
            pl.BlockSpec((_MB, D), lambda t, j: (j, 0)),
            pl.BlockSpec((1, D, D), lambda t, j: (t, 0, 0)),
        ],
        out_specs=pl.BlockSpec(
            (_MB, D), lambda t, j: (t * (N_NODES // _MB) + j, 0)),
        out_shape=jax.ShapeDtypeStruct((N_REL * N_NODES, D), _f32),
    )(x, weights)


def _fin_body(p0_ref, p1_ref, b_ref, o_ref):
    s = p0_ref[...] + p1_ref[...] + b_ref[...]
    o_ref[...] = jnp.where(s >= 0, s, 0.01 * s)


def _finish(partial, bias):
    nb = N_NODES // _MB
    return pl.pallas_call(
        _fin_body,
        grid=(nb,),
        in_specs=[
            pl.BlockSpec((_MB, D), lambda j: (j, 0)),
            pl.BlockSpec((_MB, D), lambda j: (j + nb, 0)),
            pl.BlockSpec((1, D), lambda j: (0, 0)),
        ],
        out_specs=pl.BlockSpec((_MB, D), lambda j: (j, 0)),
        out_shape=jax.ShapeDtypeStruct((N_NODES, D), _f32),
    )(partial, partial, bias)


def kernel(x, edge_index, edge_type, edge_weight, weights, bias):
    row2d = edge_index[0].reshape(ROWS2D, K)
    col2d = edge_index[1].reshape(ROWS2D, K)
    typ2d = edge_type.reshape(ROWS2D, K)
    ew2d = edge_weight.reshape(ROWS2D, K)
    table = _project(x, weights)
    partial = _get_sc_kernel()(row2d, col2d, typ2d, ew2d, table)
    return _finish(partial, bias.reshape(1, D))


# speedup vs baseline: 44.6157x; 1.3384x over previous
import functools

import jax
import jax.numpy as jnp
from jax import lax
from jax.experimental import pallas as pl
from jax.experimental.pallas import tpu as pltpu
from jax.experimental.pallas import tpu_sc as plsc

N_NODES = 10000
N_EDGES = 320000
D = 128
N_REL = 4

NC = 2
NS = 16
NW = NC * NS

K = 80
ROWS2D = N_EDGES // K
CH_T = N_EDGES // NW // K
BS = 25
HROWS = ROWS2D // NS
DROWS = 640

_f32 = jnp.float32
_i32 = jnp.int32


def _zero16():
    return jnp.zeros((16,), _f32)


def _rsqrt16(d):
    i = plsc.bitcast(d, _i32)
    i = 0x5F3759DF - lax.shift_right_logical(i, 1)
    y = plsc.bitcast(i, _f32)
    half_d = 0.5 * d
    for _ in range(3):
        y = y * (1.5 - half_d * y * y)
    return jnp.where(d > 0.5, y, jnp.zeros_like(y))


def _sc_body(row_hbm, col_hbm, typ_hbm, ew_hbm, table_hbm, out_hbm,
             row_s, col_s, typ_s, ew_s, dnv, idx2, buf0, buf1, buf2,
             gs0, gs1, gs2, cs0, cs1, cs2, deg_sh, acc_sh):
    cid = lax.axis_index("c")
    sid = lax.axis_index("s")
    wid = cid * NS + sid
    ones16 = jnp.ones((16,), _f32)

    def _z_dnv(i, _):
        dnv[i, :] = _zero16()
        return _
    lax.fori_loop(0, DROWS, _z_dnv, None)

    def _z_buf(i, _):
        for q in range(D // 16):
            buf0[i, pl.ds(q * 16, 16)] = _zero16()
        return _
    lax.fori_loop(0, K, _z_buf, None)

    def _z_idx(k, _):
        for q in range(8):
            idx2[k, pl.ds(q * 16, 16)] = (
                lax.iota(_i32, 16) + (k * 128 + q * 16))
        return _
    lax.fori_loop(0, 5, _z_idx, None)

    def _z_acc(k, _):
        pltpu.sync_copy(buf0, acc_sh.at[pl.ds(sid * 625 + k * K, K)])
        return _
    lax.fori_loop(0, 7, _z_acc, None)
    pltpu.sync_copy(buf0.at[pl.ds(0, 65)],
                    acc_sh.at[pl.ds(sid * 625 + 7 * K, 65)])

    @pl.when(sid == 0)
    def _():
        pltpu.sync_copy(dnv, deg_sh)

    plsc.subcore_barrier()

    def _hist_stage(st, _):
        pltpu.sync_copy(col_hbm.at[pl.ds(sid * HROWS + st * BS, BS)], col_s)

        def _hrow(j, _h):
            for q in range(K // 16):
                v = col_s[j, pl.ds(q * 16, 16)]
                plsc.addupdate_scatter(
                    dnv, [lax.shift_right_logical(v, 4),
                          jnp.bitwise_and(v, 15)], ones16)
            return _h
        lax.fori_loop(0, BS, _hrow, None)
        return _
    lax.fori_loop(0, HROWS // BS, _hist_stage, None)

    for k in range(5):
        pltpu.sync_copy(dnv.at[pl.ds(k * 128, 128)],
                        deg_sh.at[idx2.at[k]], add=True)
    plsc.subcore_barrier()

    pltpu.sync_copy(deg_sh, dnv)

    def _dinv(i, _):
        dnv[i, :] = _rsqrt16(dnv[i, :])
        return _
    lax.fori_loop(0, DROWS, _dinv, None)

    def _scale(buf, ref, j):
        def _sgrp(q, _):
            sv16 = ref[j, pl.ds(q * 16, 16)]
            for l in range(16):
                sv = jnp.full((16,), sv16[l], _f32)
                i = q * 16 + l
                for p in range(D // 16):
                    sl = pl.ds(p * 16, 16)
                    buf[i, sl] = buf[i, sl] * sv
            return _
        lax.fori_loop(0, K // 16, _sgrp, None)

    def _block(b, _):
        base = wid * CH_T + b * BS
        pltpu.sync_copy(row_hbm.at[pl.ds(base, BS)], row_s)
        pltpu.sync_copy(col_hbm.at[pl.ds(base, BS)], col_s)
        pltpu.sync_copy(typ_hbm.at[pl.ds(base, BS)], typ_s)
        pltpu.sync_copy(ew_hbm.at[pl.ds(base, BS)], ew_s)

        def _prep(j, _p):
            for q in range(K // 16):
                sl = pl.ds(q * 16, 16)
                r = row_s[j, sl]
                c = col_s[j, sl]
                t = typ_s[j, sl]
                w = ew_s[j, sl]
                dr = plsc.load_gather(
                    dnv, [lax.shift_right_logical(r, 4),
                          jnp.bitwise_and(r, 15)])
                dc = plsc.load_gather(
                    dnv, [lax.shift_right_logical(c, 4),
                          jnp.bitwise_and(c, 15)])
                ew_s[j, sl] = w * dr * dc
                col_s[j, sl] = t * N_NODES + c
            return _p
        lax.fori_loop(0, BS, _prep, None)

        def _g(j, buf, sem):
            return pltpu.async_copy(table_hbm.at[col_s.at[j]], buf, sem)

        def _sc(j, buf, sem):
            return pltpu.async_copy(buf, acc_sh.at[row_s.at[j]], sem,
                                    add=True)

        def _scdrain(buf, sem):
            pltpu.make_async_copy(buf, acc_sh.at[row_s.at[0]], sem).wait()

        _g(0, buf0, gs0)
        _g(1, buf1, gs1)

        def _tri(p, _p):
            j0 = 3 * p
            pltpu.make_async_copy(table_hbm.at[col_s.at[j0]], buf0,
                                  gs0).wait()
            _scale(buf0, ew_s, j0)
            d0 = _sc(j0, buf0, cs0)
            @pl.when(p > 0)
            def _():
                _scdrain(buf2, cs2)
            _g(j0 + 2, buf2, gs2)

            pltpu.make_async_copy(table_hbm.at[col_s.at[j0 + 1]], buf1,
                                  gs1).wait()
            _scale(buf1, ew_s, j0 + 1)
            d1 = _sc(j0 + 1, buf1, cs1)
            d0.wait()
            _g(j0 + 3, buf0, gs0)

            pltpu.make_async_copy(table_hbm.at[col_s.at[j0 + 2]], buf2,
                                  gs2).wait()
            _scale(buf2, ew_s, j0 + 2)
            _sc(j0 + 2, buf2, cs2)

            @pl.when(p < (BS // 3) - 1)
            def _():
                d1.wait()
                _g(j0 + 4, buf1, gs1)
            return _p
        lax.fori_loop(0, BS // 3, _tri, None)

        jl = BS - 1
        pltpu.make_async_copy(table_hbm.at[col_s.at[jl]], buf0, gs0).wait()
        _scale(buf0, ew_s, jl)
        _sc(jl, buf0, cs0)
        _scdrain(buf1, cs1)
        _scdrain(buf2, cs2)
        _scdrain(buf0, cs0)
        return _
    lax.fori_loop(0, CH_T // BS, _block, None)

    plsc.subcore_barrier()

    def _out(k, _):
        rbase = sid * 625 + k * K
        pltpu.sync_copy(acc_sh.at[pl.ds(rbase, K)], buf0)
        pltpu.sync_copy(buf0, out_hbm.at[pl.ds(cid * N_NODES + rbase, K)])
        return _
    lax.fori_loop(0, 7, _out, None)
    pltpu.sync_copy(acc_sh.at[pl.ds(sid * 625 + 7 * K, 65)],
                    buf0.at[pl.ds(0, 65)])
    pltpu.sync_copy(buf0.at[pl.ds(0, 65)],
                    out_hbm.at[pl.ds(cid * N_NODES + sid * 625 + 7 * K, 65)])


def _make_sc_kernel():
    mesh = plsc.VectorSubcoreMesh(core_axis_name="c", subcore_axis_name="s")

    return pl.kernel(
        _sc_body,
        out_type=jax.ShapeDtypeStruct((NC * N_NODES, D), _f32),
        mesh=mesh,
        compiler_params=pltpu.CompilerParams(
            needs_layout_passes=False, use_tc_tiling_on_sc=False),
        scratch_types=[
            pltpu.VMEM((BS, K), _i32),
            pltpu.VMEM((BS, K), _i32),
            pltpu.VMEM((BS, K), _i32),
            pltpu.VMEM((BS, K), _f32),
            pltpu.VMEM((DROWS, 16), _f32),
            pltpu.VMEM((5, 128), _i32),
            pltpu.VMEM((K, D), _f32),
            pltpu.VMEM((K, D), _f32),
            pltpu.VMEM((K, D), _f32),
            pltpu.SemaphoreType.DMA,
            pltpu.SemaphoreType.DMA,
            pltpu.SemaphoreType.DMA,
            pltpu.SemaphoreType.DMA,
            pltpu.SemaphoreType.DMA,
            pltpu.SemaphoreType.DMA,
            pltpu.VMEM_SHARED((DROWS, 16), _f32),
            pltpu.VMEM_SHARED((N_NODES, D), _f32),
        ],
    )


_sc_kernel_cache = []


def _get_sc_kernel():
    if not _sc_kernel_cache:
        _sc_kernel_cache.append(_make_sc_kernel())
    return _sc_kernel_cache[0]


def _mm_body(x_ref, w_ref, o_ref):
    o_ref[...] = jnp.dot(x_ref[...], w_ref[0],
                         preferred_element_type=_f32)


_MB = 1000


def _project(x, weights):
    return pl.pallas_call(
        _mm_body,
        grid=(N_REL, N_NODES // _MB),
        in_specs=[
            pl.BlockSpec((_MB, D), lambda t, j: (j, 0)),
            pl.BlockSpec((1, D, D), lambda t, j: (t, 0, 0)),
        ],
        out_specs=pl.BlockSpec(
            (_MB, D), lambda t, j: (t * (N_NODES // _MB) + j, 0)),
        out_shape=jax.ShapeDtypeStruct((N_REL * N_NODES, D), _f32),
    )(x, weights)


def _fin_body(p0_ref, p1_ref, b_ref, o_ref):
    s = p0_ref[...] + p1_ref[...] + b_ref[...]
    o_ref[...] = jnp.where(s >= 0, s, 0.01 * s)


def _finish(partial, bias):
    nb = N_NODES // _MB
    return pl.pallas_call(
        _fin_body,
        grid=(nb,),
        in_specs=[
            pl.BlockSpec((_MB, D), lambda j: (j, 0)),
            pl.BlockSpec((_MB, D), lambda j: (j + nb, 0)),
            pl.BlockSpec((1, D), lambda j: (0, 0)),
        ],
        out_specs=pl.BlockSpec((_MB, D), lambda j: (j, 0)),
        out_shape=jax.ShapeDtypeStruct((N_NODES, D), _f32),
    )(partial, partial, bias)


def kernel(x, edge_index, edge_type, edge_weight, weights, bias):
    row2d = edge_index[0].reshape(ROWS2D, K)
    col2d = edge_index[1].reshape(ROWS2D, K)
    typ2d = edge_type.reshape(ROWS2D, K)
    ew2d = edge_weight.reshape(ROWS2D, K)
    table = _project(x, weights)
    partial = _get_sc_kernel()(row2d, col2d, typ2d, ew2d, table)
    return _finish(partial, bias.reshape(1, D))
